# baseline (speedup 1.0000x reference)
import functools

import jax
import jax.numpy as jnp
from jax import lax
from jax.experimental import pallas as pl
from jax.experimental.pallas import tpu as pltpu
from jax.experimental.pallas import tpu_sc as plsc

B, N, F, D, K = 4, 10000, 64, 64, 20
BN = B * N
NW = 32
CH = 16
NCH = BN // CH
CPW = (NCH + NW - 1) // NW
KP1 = K + 1
NF = D // 16


def _gat_body(topk_t, xl, ad, asrc, out, as_v, tk_v, idx_v, attn_v,
              rows_v, ad_v, out_v, sem):
    wid = lax.axis_index("s") * 2 + lax.axis_index("c")
    pltpu.sync_copy(asrc, as_v)

    def chunk_body(t, carry):
        c = wid + t * NW

        @pl.when(c < NCH)
        def _():
            g0 = c * CH
            b = g0 // N
            i0 = g0 - b * N
            off = b * N
            cc = i0 // CH
            pltpu.sync_copy(topk_t.at[pl.ds(cc * (K * CH), K * CH)], tk_v)
            pltpu.sync_copy(ad.at[pl.ds(g0, CH)], ad_v)
            lane = lax.iota(jnp.int32, CH)
            adv = ad_v[...]
            alphas = []
            for k in range(K):
                tkk = tk_v[pl.ds(k * CH, CH)]
                gk = tkk + off
                j, p = divmod(k, 7)
                idx_v[j, pl.ds(p * CH, CH)] = gk
                a = adv + plsc.load_gather(as_v, [gk])
                a = jnp.where(a >= 0, a, 0.2 * a)
                a = jnp.where(tkk == i0 + lane, -1e30, a)
                alphas.append(a)
            gs = g0 + lane
            idx_v[2, pl.ds(6 * CH, CH)] = gs
            a = adv + as_v[pl.ds(g0, CH)]
            alphas.append(jnp.where(a >= 0, a, 0.2 * a))
            cps = [pltpu.async_copy(xl.at[idx_v.at[j]],
                                    rows_v.at[pl.ds(j * 7 * CH, 7 * CH)], sem)
                   for j in range(3)]
            m = alphas[0]
            for a in alphas[1:]:
                m = jnp.maximum(m, a)
            es = [jnp.exp(a - m) for a in alphas]
            ssum = es[0]
            for e in es[1:]:
                ssum = ssum + e
            rcp = 1.0 / (ssum + 1e-16)
            for k, e in enumerate(es):
                attn_v[pl.ds(k * CH, CH)] = e * rcp
            for cp in cps:
                cp.wait()

            def node_body(n, carry2):
                accs = [jnp.zeros((16,), jnp.float32) for _ in range(NF)]
                for k in range(KP1):
                    wgt = plsc.load_gather(
                        attn_v, [jnp.full((CH,), k * CH + n, jnp.int32)])
                    r = k * CH + n
                    for f in range(NF):
                        accs[f] = accs[f] + wgt * rows_v[r, pl.ds(f * 16, 16)]
                for f in range(NF):
                    out_v[n, pl.ds(f * 16, 16)] = accs[f]
                return carry2

            lax.fori_loop(0, CH, node_body, 0)
            pltpu.sync_copy(out_v, out.at[pl.ds(g0, CH), :])

        return carry

    lax.fori_loop(0, CPW, chunk_body, 0)


def _gat_aggregate(topk_t, xl, ad, asrc):
    mesh = plsc.VectorSubcoreMesh(core_axis_name="c", subcore_axis_name="s")
    kfn = pl.kernel(
        _gat_body,
        out_type=jax.ShapeDtypeStruct((BN, D), jnp.float32),
        mesh=mesh,
        compiler_params=pltpu.CompilerParams(
            needs_layout_passes=False, use_tc_tiling_on_sc=False),
        scratch_types=[
            pltpu.VMEM((BN,), jnp.float32),
            pltpu.VMEM((K * CH,), jnp.int32),
            pltpu.VMEM((3, 7 * CH), jnp.int32),
            pltpu.VMEM((KP1 * CH,), jnp.float32),
            pltpu.VMEM((KP1 * CH, D), jnp.float32),
            pltpu.VMEM((CH,), jnp.float32),
            pltpu.VMEM((CH, D), jnp.float32),
            pltpu.SemaphoreType.DMA,
        ],
    )
    return kfn(topk_t, xl, ad, asrc)


TOPK_R = 256
NPAD = 10240


def _topk_body(wblk_ref, wall_ref, rs_ref, idx_ref):
    s = jax.lax.dot_general(
        wblk_ref[...], wall_ref[...], (((1,), (1,)), ((), ())),
        preferred_element_type=jnp.float32)
    s = s * rs_ref[...]
    ii = jax.lax.broadcasted_iota(jnp.int32, (TOPK_R, N), 1)
    m = jnp.max(s, axis=1, keepdims=True)
    for k in range(K):
        am = jnp.min(jnp.where(s == m, ii, N), axis=1, keepdims=True)
        idx_ref[:, k:k + 1] = am
        if k < K - 1:
            m = jnp.max(jnp.where(s >= m, -jnp.inf, s), axis=1,
                        keepdims=True)


def _cos_topk(w):
    rs = jax.lax.rsqrt(jnp.sum(w * w, axis=1)).reshape(1, N)
    wpad = jnp.zeros((NPAD, D), jnp.float32).at[:N].set(w)
    idx = pl.pallas_call(
        _topk_body,
        grid=(NPAD // TOPK_R,),
        in_specs=[
            pl.BlockSpec((TOPK_R, D), lambda i: (i, 0)),
            pl.BlockSpec((N, D), lambda i: (0, 0)),
            pl.BlockSpec((1, N), lambda i: (0, 0)),
        ],
        out_specs=pl.BlockSpec((TOPK_R, K), lambda i: (i, 0)),
        out_shape=jax.ShapeDtypeStruct((NPAD, K), jnp.int32),
    )(wpad, w, rs)
    return idx[:N]


TB = 2000


def _stats_body(x_ref, st_ref):
    @pl.when(pl.program_id(0) == 0)
    def _():
        st_ref[...] = jnp.zeros_like(st_ref)
    x = x_ref[...]
    st_ref[0:1, :] += jnp.sum(x, axis=0, keepdims=True)
    st_ref[1:2, :] += jnp.sum(x * x, axis=0, keepdims=True)


def _stats(x):
    return pl.pallas_call(
        _stats_body, grid=(BN // TB,),
        in_specs=[pl.BlockSpec((TB, D), lambda i: (i, 0))],
        out_specs=pl.BlockSpec((2, D), lambda i: (0, 0)),
        out_shape=jax.ShapeDtypeStruct((2, D), jnp.float32))(x)


def _bn_mul_body(x_ref, emb_ref, a_ref, b_ref, z_ref, st_ref):
    y = jnp.maximum(x_ref[...] * a_ref[...] + b_ref[...], 0.0)
    z = y * emb_ref[...]
    z_ref[...] = z
    @pl.when(pl.program_id(0) == 0)
    def _():
        st_ref[...] = jnp.zeros_like(st_ref)
    st_ref[0:1, :] += jnp.sum(z, axis=0, keepdims=True)
    st_ref[1:2, :] += jnp.sum(z * z, axis=0, keepdims=True)


def _bn_mul(x, emb, a, b):
    return pl.pallas_call(
        _bn_mul_body, grid=(BN // TB,),
        in_specs=[pl.BlockSpec((TB, D), lambda i: (i, 0)),
                  pl.BlockSpec((TB, D), lambda i: (i % (N // TB), 0)),
                  pl.BlockSpec((1, D), lambda i: (0, 0)),
                  pl.BlockSpec((1, D), lambda i: (0, 0))],
        out_specs=[pl.BlockSpec((TB, D), lambda i: (i, 0)),
                   pl.BlockSpec((2, D), lambda i: (0, 0))],
        out_shape=[jax.ShapeDtypeStruct((BN, D), jnp.float32),
                   jax.ShapeDtypeStruct((2, D), jnp.float32)])(x, emb, a, b)


def _bn_out_body(z_ref, a_ref, b_ref, wv_ref, o_ref):
    q = jnp.maximum(z_ref[...] * a_ref[...] + b_ref[...], 0.0)
    o_ref[...] = jnp.sum(q * wv_ref[...], axis=1, keepdims=True)


def _bn_out(z, a, b, wv):
    return pl.pallas_call(
        _bn_out_body, grid=(BN // TB,),
        in_specs=[pl.BlockSpec((TB, D), lambda i: (i, 0)),
                  pl.BlockSpec((1, D), lambda i: (0, 0)),
                  pl.BlockSpec((1, D), lambda i: (0, 0)),
                  pl.BlockSpec((1, D), lambda i: (0, 0))],
        out_specs=pl.BlockSpec((TB, 1), lambda i: (i, 0)),
        out_shape=jax.ShapeDtypeStruct((BN, 1), jnp.float32))(z, a, b, wv)


def _proj_body(x_ref, w_ref, o_ref):
    o_ref[...] = jnp.dot(x_ref[...], w_ref[...],
                         preferred_element_type=jnp.float32)


def _project(x, lin_W):
    blk = 2000
    return pl.pallas_call(
        _proj_body,
        grid=(BN // blk,),
        in_specs=[
            pl.BlockSpec((blk, F), lambda i: (i, 0)),
            pl.BlockSpec((F, D), lambda i: (0, 0)),
        ],
        out_specs=pl.BlockSpec((blk, D), lambda i: (i, 0)),
        out_shape=jax.ShapeDtypeStruct((BN, D), jnp.float32),
    )(x, lin_W)


def kernel(data, org_edge_index, emb_weight, lin_W, att_i, att_j, att_em_i,
           att_em_j, gnn_bias, bn1_gamma, bn1_beta, bn2_gamma, bn2_beta,
           out_W, out_b):
    w = emb_weight
    topk_idx = _cos_topk(w)

    x = data.reshape(-1, F)
    xl = _project(x, lin_W)

    emb_ai = w @ att_em_i
    emb_aj = w @ att_em_j
    ad = ((xl @ att_i).reshape(B, N) + emb_ai[None, :]).reshape(BN)
    asrc = ((xl @ att_j).reshape(B, N) + emb_aj[None, :]).reshape(BN)

    topk_t = topk_idx.reshape(N // CH, CH, K).transpose(0, 2, 1).reshape(-1)
    out = _gat_aggregate(topk_t, xl, ad, asrc)

    st1 = _stats(out)
    mu1 = st1[0] / BN
    var1 = st1[1] / BN - mu1 * mu1
    a1 = bn1_gamma * jax.lax.rsqrt(var1 + 1e-5)
    c1 = bn1_beta - mu1 * a1
    z, st2 = _bn_mul(out, w, a1.reshape(1, D), c1.reshape(1, D))
    mu2 = st2[0] / BN
    var2 = st2[1] / BN - mu2 * mu2
    a2 = bn2_gamma * jax.lax.rsqrt(var2 + 1e-5)
    c2 = bn2_beta - mu2 * a2
    o = _bn_out(z, a2.reshape(1, D), c2.reshape(1, D), out_W.reshape(1, D))
    return (o + out_b).reshape(-1, N)


# speedup vs baseline: 1.0652x; 1.0652x over previous
import functools

import jax
import jax.numpy as jnp
from jax import lax
from jax.experimental import pallas as pl
from jax.experimental.pallas import tpu as pltpu
from jax.experimental.pallas import tpu_sc as plsc

B, N, F, D, K = 4, 10000, 64, 64, 20
BN = B * N
NW = 32
CH = 16
NCH = BN // CH
CPW = (NCH + NW - 1) // NW
KP1 = K + 1
NF = D // 16


def _gat_body(topk_t, xl, ad, asrc, out, as_v, tk_v, idx_v, attn_v,
              rows_v, ad_v, out_v, sem):
    wid = lax.axis_index("s") * 2 + lax.axis_index("c")
    pltpu.sync_copy(asrc, as_v)

    def chunk_body(t, carry):
        c = wid + t * NW

        @pl.when(c < NCH)
        def _():
            g0 = c * CH
            b = g0 // N
            i0 = g0 - b * N
            off = b * N
            cc = i0 // CH
            pltpu.sync_copy(topk_t.at[pl.ds(cc * (K * CH), K * CH)], tk_v)
            pltpu.sync_copy(ad.at[pl.ds(g0, CH)], ad_v)
            lane = lax.iota(jnp.int32, CH)
            adv = ad_v[...]
            alphas = []
            for k in range(K):
                tkk = tk_v[pl.ds(k * CH, CH)]
                gk = tkk + off
                j, p = divmod(k, 7)
                idx_v[j, pl.ds(p * CH, CH)] = gk
                a = adv + plsc.load_gather(as_v, [gk])
                a = jnp.where(a >= 0, a, 0.2 * a)
                a = jnp.where(tkk == i0 + lane, -1e30, a)
                alphas.append(a)
            gs = g0 + lane
            idx_v[2, pl.ds(6 * CH, CH)] = gs
            a = adv + as_v[pl.ds(g0, CH)]
            alphas.append(jnp.where(a >= 0, a, 0.2 * a))
            cps = [pltpu.async_copy(xl.at[idx_v.at[j]],
                                    rows_v.at[pl.ds(j * 7 * CH, 7 * CH)], sem)
                   for j in range(3)]
            m = alphas[0]
            for a in alphas[1:]:
                m = jnp.maximum(m, a)
            es = [jnp.exp(a - m) for a in alphas]
            ssum = es[0]
            for e in es[1:]:
                ssum = ssum + e
            rcp = 1.0 / (ssum + 1e-16)
            for k, e in enumerate(es):
                attn_v[pl.ds(k * CH, CH)] = e * rcp
            for cp in cps:
                cp.wait()

            def node_body(n, carry2):
                accs = [jnp.zeros((16,), jnp.float32) for _ in range(NF)]
                for k in range(KP1):
                    wgt = plsc.load_gather(
                        attn_v, [jnp.full((CH,), k * CH + n, jnp.int32)])
                    r = k * CH + n
                    for f in range(NF):
                        accs[f] = accs[f] + wgt * rows_v[r, pl.ds(f * 16, 16)]
                for f in range(NF):
                    out_v[n, pl.ds(f * 16, 16)] = accs[f]
                return carry2

            lax.fori_loop(0, CH, node_body, 0)
            pltpu.sync_copy(out_v, out.at[pl.ds(g0, CH), :])

        return carry

    lax.fori_loop(0, CPW, chunk_body, 0)


def _gat_aggregate(topk_t, xl, ad, asrc):
    mesh = plsc.VectorSubcoreMesh(core_axis_name="c", subcore_axis_name="s")
    kfn = pl.kernel(
        _gat_body,
        out_type=jax.ShapeDtypeStruct((BN, D), jnp.float32),
        mesh=mesh,
        compiler_params=pltpu.CompilerParams(
            needs_layout_passes=False, use_tc_tiling_on_sc=False),
        scratch_types=[
            pltpu.VMEM((BN,), jnp.float32),
            pltpu.VMEM((K * CH,), jnp.int32),
            pltpu.VMEM((3, 7 * CH), jnp.int32),
            pltpu.VMEM((KP1 * CH,), jnp.float32),
            pltpu.VMEM((KP1 * CH, D), jnp.float32),
            pltpu.VMEM((CH,), jnp.float32),
            pltpu.VMEM((CH, D), jnp.float32),
            pltpu.SemaphoreType.DMA,
        ],
    )
    return kfn(topk_t, xl, ad, asrc)


TOPK_R = 400
NPAD = 10000


def _topk_body(wblk_ref, wall_ref, rs_ref, idx_ref):
    s = jax.lax.dot_general(
        wblk_ref[...], wall_ref[...], (((1,), (1,)), ((), ())),
        preferred_element_type=jnp.float32)
    s = s * rs_ref[...]
    ii = jax.lax.broadcasted_iota(jnp.int32, (TOPK_R, N), 1)
    m = jnp.max(s, axis=1, keepdims=True)
    for k in range(K):
        am = jnp.min(jnp.where(s == m, ii, N), axis=1, keepdims=True)
        idx_ref[:, k:k + 1] = am
        if k < K - 1:
            m = jnp.max(jnp.where(s >= m, -jnp.inf, s), axis=1,
                        keepdims=True)


def _cos_topk(w):
    rs = jax.lax.rsqrt(jnp.sum(w * w, axis=1)).reshape(1, N)
    if NPAD == N:
        wpad = w
    else:
        wpad = jnp.zeros((NPAD, D), jnp.float32).at[:N].set(w)
    idx = pl.pallas_call(
        _topk_body,
        grid=(NPAD // TOPK_R,),
        in_specs=[
            pl.BlockSpec((TOPK_R, D), lambda i: (i, 0)),
            pl.BlockSpec((N, D), lambda i: (0, 0)),
            pl.BlockSpec((1, N), lambda i: (0, 0)),
        ],
        out_specs=pl.BlockSpec((TOPK_R, K), lambda i: (i, 0)),
        out_shape=jax.ShapeDtypeStruct((NPAD, K), jnp.int32),
    )(wpad, w, rs)
    return idx[:N]


TB = 2000


def _stats_body(x_ref, st_ref):
    @pl.when(pl.program_id(0) == 0)
    def _():
        st_ref[...] = jnp.zeros_like(st_ref)
    x = x_ref[...]
    st_ref[0:1, :] += jnp.sum(x, axis=0, keepdims=True)
    st_ref[1:2, :] += jnp.sum(x * x, axis=0, keepdims=True)


def _stats(x):
    return pl.pallas_call(
        _stats_body, grid=(BN // TB,),
        in_specs=[pl.BlockSpec((TB, D), lambda i: (i, 0))],
        out_specs=pl.BlockSpec((2, D), lambda i: (0, 0)),
        out_shape=jax.ShapeDtypeStruct((2, D), jnp.float32))(x)


def _bn_mul_body(x_ref, emb_ref, a_ref, b_ref, z_ref, st_ref):
    y = jnp.maximum(x_ref[...] * a_ref[...] + b_ref[...], 0.0)
    z = y * emb_ref[...]
    z_ref[...] = z
    @pl.when(pl.program_id(0) == 0)
    def _():
        st_ref[...] = jnp.zeros_like(st_ref)
    st_ref[0:1, :] += jnp.sum(z, axis=0, keepdims=True)
    st_ref[1:2, :] += jnp.sum(z * z, axis=0, keepdims=True)


def _bn_mul(x, emb, a, b):
    return pl.pallas_call(
        _bn_mul_body, grid=(BN // TB,),
        in_specs=[pl.BlockSpec((TB, D), lambda i: (i, 0)),
                  pl.BlockSpec((TB, D), lambda i: (i % (N // TB), 0)),
                  pl.BlockSpec((1, D), lambda i: (0, 0)),
                  pl.BlockSpec((1, D), lambda i: (0, 0))],
        out_specs=[pl.BlockSpec((TB, D), lambda i: (i, 0)),
                   pl.BlockSpec((2, D), lambda i: (0, 0))],
        out_shape=[jax.ShapeDtypeStruct((BN, D), jnp.float32),
                   jax.ShapeDtypeStruct((2, D), jnp.float32)])(x, emb, a, b)


def _bn_out_body(z_ref, a_ref, b_ref, wv_ref, o_ref):
    q = jnp.maximum(z_ref[...] * a_ref[...] + b_ref[...], 0.0)
    o_ref[...] = jnp.sum(q * wv_ref[...], axis=1, keepdims=True)


def _bn_out(z, a, b, wv):
    return pl.pallas_call(
        _bn_out_body, grid=(BN // TB,),
        in_specs=[pl.BlockSpec((TB, D), lambda i: (i, 0)),
                  pl.BlockSpec((1, D), lambda i: (0, 0)),
                  pl.BlockSpec((1, D), lambda i: (0, 0)),
                  pl.BlockSpec((1, D), lambda i: (0, 0))],
        out_specs=pl.BlockSpec((TB, 1), lambda i: (i, 0)),
        out_shape=jax.ShapeDtypeStruct((BN, 1), jnp.float32))(z, a, b, wv)


def _proj_body(x_ref, w_ref, o_ref):
    o_ref[...] = jnp.dot(x_ref[...], w_ref[...],
                         preferred_element_type=jnp.float32)


def _project(x, lin_W):
    blk = 2000
    return pl.pallas_call(
        _proj_body,
        grid=(BN // blk,),
        in_specs=[
            pl.BlockSpec((blk, F), lambda i: (i, 0)),
            pl.BlockSpec((F, D), lambda i: (0, 0)),
        ],
        out_specs=pl.BlockSpec((blk, D), lambda i: (i, 0)),
        out_shape=jax.ShapeDtypeStruct((BN, D), jnp.float32),
    )(x, lin_W)


def kernel(data, org_edge_index, emb_weight, lin_W, att_i, att_j, att_em_i,
           att_em_j, gnn_bias, bn1_gamma, bn1_beta, bn2_gamma, bn2_beta,
           out_W, out_b):
    w = emb_weight
    topk_idx = _cos_topk(w)

    x = data.reshape(-1, F)
    xl = _project(x, lin_W)

    emb_ai = w @ att_em_i
    emb_aj = w @ att_em_j
    ad = ((xl @ att_i).reshape(B, N) + emb_ai[None, :]).reshape(BN)
    asrc = ((xl @ att_j).reshape(B, N) + emb_aj[None, :]).reshape(BN)

    topk_t = topk_idx.reshape(N // CH, CH, K).transpose(0, 2, 1).reshape(-1)
    out = _gat_aggregate(topk_t, xl, ad, asrc)

    st1 = _stats(out)
    mu1 = st1[0] / BN
    var1 = st1[1] / BN - mu1 * mu1
    a1 = bn1_gamma * jax.lax.rsqrt(var1 + 1e-5)
    c1 = bn1_beta - mu1 * a1
    z, st2 = _bn_mul(out, w, a1.reshape(1, D), c1.reshape(1, D))
    mu2 = st2[0] / BN
    var2 = st2[1] / BN - mu2 * mu2
    a2 = bn2_gamma * jax.lax.rsqrt(var2 + 1e-5)
    c2 = bn2_beta - mu2 * a2
    o = _bn_out(z, a2.reshape(1, D), c2.reshape(1, D), out_W.reshape(1, D))
    return (o + out_b).reshape(-1, N)


# speedup vs baseline: 1.1068x; 1.0391x over previous
import functools

import jax
import jax.numpy as jnp
from jax import lax
from jax.experimental import pallas as pl
from jax.experimental.pallas import tpu as pltpu
from jax.experimental.pallas import tpu_sc as plsc

B, N, F, D, K = 4, 10000, 64, 64, 20
BN = B * N
NW = 32
CH = 16
NCH = BN // CH
CPW = (NCH + NW - 1) // NW
KP1 = K + 1
NF = D // 16


def _gat_body(topk_t, xl, ad, asrc, out, as_v, tk_v, idx_v, attn_v,
              rows_v, ad_v, out_v, sem):
    wid = lax.axis_index("s") * 2 + lax.axis_index("c")
    pltpu.sync_copy(asrc, as_v)

    def chunk_body(t, carry):
        c = wid + t * NW

        @pl.when(c < NCH)
        def _():
            g0 = c * CH
            b = g0 // N
            i0 = g0 - b * N
            off = b * N
            cc = i0 // CH
            pltpu.sync_copy(topk_t.at[pl.ds(cc * (K * CH), K * CH)], tk_v)
            pltpu.sync_copy(ad.at[pl.ds(g0, CH)], ad_v)
            lane = lax.iota(jnp.int32, CH)
            adv = ad_v[...]
            alphas = []
            for k in range(K):
                tkk = tk_v[pl.ds(k * CH, CH)]
                gk = tkk + off
                j, p = divmod(k, 7)
                idx_v[j, pl.ds(p * CH, CH)] = gk
                a = adv + plsc.load_gather(as_v, [gk])
                a = jnp.where(a >= 0, a, 0.2 * a)
                a = jnp.where(tkk == i0 + lane, -1e30, a)
                alphas.append(a)
            gs = g0 + lane
            idx_v[2, pl.ds(6 * CH, CH)] = gs
            a = adv + as_v[pl.ds(g0, CH)]
            alphas.append(jnp.where(a >= 0, a, 0.2 * a))
            cps = [pltpu.async_copy(xl.at[idx_v.at[j]],
                                    rows_v.at[pl.ds(j * 7 * CH, 7 * CH)], sem)
                   for j in range(3)]
            m = alphas[0]
            for a in alphas[1:]:
                m = jnp.maximum(m, a)
            es = [jnp.exp(a - m) for a in alphas]
            ssum = es[0]
            for e in es[1:]:
                ssum = ssum + e
            rcp = 1.0 / (ssum + 1e-16)
            for k, e in enumerate(es):
                attn_v[pl.ds(k * CH, CH)] = e * rcp
            for cp in cps:
                cp.wait()

            def node_body(n, carry2):
                accs = [jnp.zeros((16,), jnp.float32) for _ in range(NF)]
                for k in range(KP1):
                    wgt = plsc.load_gather(
                        attn_v, [jnp.full((CH,), k * CH + n, jnp.int32)])
                    r = k * CH + n
                    for f in range(NF):
                        accs[f] = accs[f] + wgt * rows_v[r, pl.ds(f * 16, 16)]
                for f in range(NF):
                    out_v[n, pl.ds(f * 16, 16)] = accs[f]
                return carry2

            lax.fori_loop(0, CH, node_body, 0)
            pltpu.sync_copy(out_v, out.at[pl.ds(g0, CH), :])

        return carry

    lax.fori_loop(0, CPW, chunk_body, 0)


def _gat_aggregate(topk_t, xl, ad, asrc):
    mesh = plsc.VectorSubcoreMesh(core_axis_name="c", subcore_axis_name="s")
    kfn = pl.kernel(
        _gat_body,
        out_type=jax.ShapeDtypeStruct((BN, D), jnp.float32),
        mesh=mesh,
        compiler_params=pltpu.CompilerParams(
            needs_layout_passes=False, use_tc_tiling_on_sc=False),
        scratch_types=[
            pltpu.VMEM((BN,), jnp.float32),
            pltpu.VMEM((K * CH,), jnp.int32),
            pltpu.VMEM((3, 7 * CH), jnp.int32),
            pltpu.VMEM((KP1 * CH,), jnp.float32),
            pltpu.VMEM((KP1 * CH, D), jnp.float32),
            pltpu.VMEM((CH,), jnp.float32),
            pltpu.VMEM((CH, D), jnp.float32),
            pltpu.SemaphoreType.DMA,
        ],
    )
    return kfn(topk_t, xl, ad, asrc)


TOPK_R = 1000
NPAD = 10000


def _topk_body(wblk_ref, wall_ref, rs_ref, idx_ref):
    s = jax.lax.dot_general(
        wblk_ref[...], wall_ref[...], (((1,), (1,)), ((), ())),
        preferred_element_type=jnp.float32)
    s = s * rs_ref[...]
    ii = jax.lax.broadcasted_iota(jnp.int32, (TOPK_R, N), 1)
    m = jnp.max(s, axis=1, keepdims=True)
    for k in range(K):
        am = jnp.min(jnp.where(s == m, ii, N), axis=1, keepdims=True)
        idx_ref[:, k:k + 1] = am
        if k < K - 1:
            m = jnp.max(jnp.where(s >= m, -jnp.inf, s), axis=1,
                        keepdims=True)


def _cos_topk(w):
    rs = jax.lax.rsqrt(jnp.sum(w * w, axis=1)).reshape(1, N)
    if NPAD == N:
        wpad = w
    else:
        wpad = jnp.zeros((NPAD, D), jnp.float32).at[:N].set(w)
    idx = pl.pallas_call(
        _topk_body,
        grid=(NPAD // TOPK_R,),
        in_specs=[
            pl.BlockSpec((TOPK_R, D), lambda i: (i, 0)),
            pl.BlockSpec((N, D), lambda i: (0, 0)),
            pl.BlockSpec((1, N), lambda i: (0, 0)),
        ],
        out_specs=pl.BlockSpec((TOPK_R, K), lambda i: (i, 0)),
        out_shape=jax.ShapeDtypeStruct((NPAD, K), jnp.int32),
    )(wpad, w, rs)
    return idx[:N]


TB = 2000


def _stats_body(x_ref, st_ref):
    @pl.when(pl.program_id(0) == 0)
    def _():
        st_ref[...] = jnp.zeros_like(st_ref)
    x = x_ref[...]
    st_ref[0:1, :] += jnp.sum(x, axis=0, keepdims=True)
    st_ref[1:2, :] += jnp.sum(x * x, axis=0, keepdims=True)


def _stats(x):
    return pl.pallas_call(
        _stats_body, grid=(BN // TB,),
        in_specs=[pl.BlockSpec((TB, D), lambda i: (i, 0))],
        out_specs=pl.BlockSpec((2, D), lambda i: (0, 0)),
        out_shape=jax.ShapeDtypeStruct((2, D), jnp.float32))(x)


def _bn_mul_body(x_ref, emb_ref, a_ref, b_ref, z_ref, st_ref):
    y = jnp.maximum(x_ref[...] * a_ref[...] + b_ref[...], 0.0)
    z = y * emb_ref[...]
    z_ref[...] = z
    @pl.when(pl.program_id(0) == 0)
    def _():
        st_ref[...] = jnp.zeros_like(st_ref)
    st_ref[0:1, :] += jnp.sum(z, axis=0, keepdims=True)
    st_ref[1:2, :] += jnp.sum(z * z, axis=0, keepdims=True)


def _bn_mul(x, emb, a, b):
    return pl.pallas_call(
        _bn_mul_body, grid=(BN // TB,),
        in_specs=[pl.BlockSpec((TB, D), lambda i: (i, 0)),
                  pl.BlockSpec((TB, D), lambda i: (i % (N // TB), 0)),
                  pl.BlockSpec((1, D), lambda i: (0, 0)),
                  pl.BlockSpec((1, D), lambda i: (0, 0))],
        out_specs=[pl.BlockSpec((TB, D), lambda i: (i, 0)),
                   pl.BlockSpec((2, D), lambda i: (0, 0))],
        out_shape=[jax.ShapeDtypeStruct((BN, D), jnp.float32),
                   jax.ShapeDtypeStruct((2, D), jnp.float32)])(x, emb, a, b)


def _bn_out_body(z_ref, a_ref, b_ref, wv_ref, o_ref):
    q = jnp.maximum(z_ref[...] * a_ref[...] + b_ref[...], 0.0)
    o_ref[...] = jnp.sum(q * wv_ref[...], axis=1, keepdims=True)


def _bn_out(z, a, b, wv):
    return pl.pallas_call(
        _bn_out_body, grid=(BN // TB,),
        in_specs=[pl.BlockSpec((TB, D), lambda i: (i, 0)),
                  pl.BlockSpec((1, D), lambda i: (0, 0)),
                  pl.BlockSpec((1, D), lambda i: (0, 0)),
                  pl.BlockSpec((1, D), lambda i: (0, 0))],
        out_specs=pl.BlockSpec((TB, 1), lambda i: (i, 0)),
        out_shape=jax.ShapeDtypeStruct((BN, 1), jnp.float32))(z, a, b, wv)


def _proj_body(x_ref, w_ref, o_ref):
    o_ref[...] = jnp.dot(x_ref[...], w_ref[...],
                         preferred_element_type=jnp.float32)


def _project(x, lin_W):
    blk = 2000
    return pl.pallas_call(
        _proj_body,
        grid=(BN // blk,),
        in_specs=[
            pl.BlockSpec((blk, F), lambda i: (i, 0)),
            pl.BlockSpec((F, D), lambda i: (0, 0)),
        ],
        out_specs=pl.BlockSpec((blk, D), lambda i: (i, 0)),
        out_shape=jax.ShapeDtypeStruct((BN, D), jnp.float32),
    )(x, lin_W)


def kernel(data, org_edge_index, emb_weight, lin_W, att_i, att_j, att_em_i,
           att_em_j, gnn_bias, bn1_gamma, bn1_beta, bn2_gamma, bn2_beta,
           out_W, out_b):
    w = emb_weight
    topk_idx = _cos_topk(w)

    x = data.reshape(-1, F)
    xl = _project(x, lin_W)

    emb_ai = w @ att_em_i
    emb_aj = w @ att_em_j
    ad = ((xl @ att_i).reshape(B, N) + emb_ai[None, :]).reshape(BN)
    asrc = ((xl @ att_j).reshape(B, N) + emb_aj[None, :]).reshape(BN)

    topk_t = topk_idx.reshape(N // CH, CH, K).transpose(0, 2, 1).reshape(-1)
    out = _gat_aggregate(topk_t, xl, ad, asrc)

    st1 = _stats(out)
    mu1 = st1[0] / BN
    var1 = st1[1] / BN - mu1 * mu1
    a1 = bn1_gamma * jax.lax.rsqrt(var1 + 1e-5)
    c1 = bn1_beta - mu1 * a1
    z, st2 = _bn_mul(out, w, a1.reshape(1, D), c1.reshape(1, D))
    mu2 = st2[0] / BN
    var2 = st2[1] / BN - mu2 * mu2
    a2 = bn2_gamma * jax.lax.rsqrt(var2 + 1e-5)
    c2 = bn2_beta - mu2 * a2
    o = _bn_out(z, a2.reshape(1, D), c2.reshape(1, D), out_W.reshape(1, D))
    return (o + out_b).reshape(-1, N)
